# baseline (speedup 1.0000x reference)
import jax
import jax.numpy as jnp
from jax.experimental import pallas as pl
from jax.experimental.pallas import tpu as pltpu

_ROWS, _WIDTH = 8192, 2048
_CH = 2048
_N = _ROWS // _CH
_K = 3
_D = 1
assert _D < _K


def _copy_kernel(src_hbm, dst_hbm, *args):
    bufs = args[:_K]
    rsem, wsem = args[_K], args[_K + 1]

    def _read(g):
        b = g % _K
        return pltpu.make_async_copy(
            src_hbm.at[pl.ds(g * _CH, _CH)], bufs[b], rsem.at[b])

    def _write(g):
        b = g % _K
        return pltpu.make_async_copy(
            bufs[b], dst_hbm.at[pl.ds(g * _CH, _CH)], wsem.at[b])

    for g in range(_N + _D):
        if g < _N:
            if g >= _K:
                _write(g - _K).wait()
            _read(g).start()
        if g >= _D:
            _read(g - _D).wait()
            _write(g - _D).start()
    for g in range(_N - _K, _N):
        _write(g).wait()


def kernel(pos_emb):
    return pl.pallas_call(
        _copy_kernel,
        out_shape=jax.ShapeDtypeStruct((_ROWS, _WIDTH), jnp.float32),
        in_specs=[pl.BlockSpec(memory_space=pl.ANY)],
        out_specs=pl.BlockSpec(memory_space=pl.ANY),
        scratch_shapes=(
            [pltpu.VMEM((_CH, _WIDTH), jnp.float32) for _ in range(_K)]
            + [pltpu.SemaphoreType.DMA((_K,)), pltpu.SemaphoreType.DMA((_K,))]
        ),
    )(pos_emb)


# speedup vs baseline: 1.0149x; 1.0149x over previous
import jax
import jax.numpy as jnp
from jax.experimental import pallas as pl
from jax.experimental.pallas import tpu as pltpu

_ROWS, _WIDTH = 8192, 2048
_CH = 2048
_N = _ROWS // _CH
_K = 3
_D = 2
assert _D < _K


def _copy_kernel(src_hbm, dst_hbm, *args):
    bufs = args[:_K]
    rsem, wsem = args[_K], args[_K + 1]

    def _read(g):
        b = g % _K
        return pltpu.make_async_copy(
            src_hbm.at[pl.ds(g * _CH, _CH)], bufs[b], rsem.at[b])

    def _write(g):
        b = g % _K
        return pltpu.make_async_copy(
            bufs[b], dst_hbm.at[pl.ds(g * _CH, _CH)], wsem.at[b])

    for g in range(_N + _D):
        if g < _N:
            if g >= _K:
                _write(g - _K).wait()
            _read(g).start()
        if g >= _D:
            _read(g - _D).wait()
            _write(g - _D).start()
    for g in range(_N - _K, _N):
        _write(g).wait()


def kernel(pos_emb):
    return pl.pallas_call(
        _copy_kernel,
        out_shape=jax.ShapeDtypeStruct((_ROWS, _WIDTH), jnp.float32),
        in_specs=[pl.BlockSpec(memory_space=pl.ANY)],
        out_specs=pl.BlockSpec(memory_space=pl.ANY),
        scratch_shapes=(
            [pltpu.VMEM((_CH, _WIDTH), jnp.float32) for _ in range(_K)]
            + [pltpu.SemaphoreType.DMA((_K,)), pltpu.SemaphoreType.DMA((_K,))]
        ),
    )(pos_emb)
